# initial kernel scaffold (unmeasured)
import jax
import jax.numpy as jnp
from jax import lax
from jax.experimental import pallas as pl
from jax.experimental.pallas import tpu as pltpu


def kernel(
    x,
):
    def body(*refs):
        pass

    out_shape = jax.ShapeDtypeStruct(..., jnp.float32)
    return pl.pallas_call(body, out_shape=out_shape)(...)


# baseline (device time: 223205 ns/iter reference)
import jax
import jax.numpy as jnp
from jax import lax
from jax.experimental import pallas as pl
from jax.experimental.pallas import tpu as pltpu

Y = 4
K = 32
CHUNK = 512
PAD = 128


def kernel(x):
    m, n = x.shape
    nsteps = n // CHUNK
    neg = float("-inf")

    def body(x_ref, out_ref, best_ref, pool_ref, mybest_ref, gather_ref,
             send_sems, recv_sems):
        step = pl.program_id(0)
        my_x = lax.axis_index("x")
        my_y = lax.axis_index("y")
        my_z = lax.axis_index("z")

        @pl.when(step == 0)
        def _init():
            best_ref[...] = jnp.full((m, PAD), neg, jnp.float32)
            gather_ref[...] = jnp.full((Y, m, K), neg, jnp.float32)

        pool_ref[:, :PAD] = best_ref[...]
        pool_ref[:, PAD:] = x_ref[...]
        best_ref[...] = jnp.full((m, PAD), neg, jnp.float32)
        iota_pad = lax.broadcasted_iota(jnp.int32, (m, PAD), 1)

        def extract(k, carry):
            pv = pool_ref[...]
            mx = jnp.max(pv, axis=1, keepdims=True)
            best_ref[...] = jnp.where(iota_pad == k, mx, best_ref[...])
            pool_ref[...] = jnp.where(pv == mx, neg, pv)
            return carry

        lax.fori_loop(0, K, extract, 0)

        @pl.when(step == nsteps - 1)
        def _comm():
            mybest_ref[...] = best_ref[:, :K]

            barrier = pltpu.get_barrier_semaphore()
            for dy in range(1, Y):
                peer = lax.rem(my_y + dy, Y)
                pl.semaphore_signal(
                    barrier, inc=1,
                    device_id=(my_x, peer, my_z),
                    device_id_type=pl.DeviceIdType.MESH,
                )
            pl.semaphore_wait(barrier, Y - 1)

            sends = []
            for dy in range(1, Y):
                peer = lax.rem(my_y + dy, Y)
                rdma = pltpu.make_async_remote_copy(
                    src_ref=mybest_ref,
                    dst_ref=gather_ref.at[my_y],
                    send_sem=send_sems.at[dy - 1],
                    recv_sem=recv_sems.at[my_y],
                    device_id=(my_x, peer, my_z),
                    device_id_type=pl.DeviceIdType.MESH,
                )
                rdma.start()
                sends.append(rdma)
            for rdma in sends:
                rdma.wait_send()
            for dy in range(1, Y):
                peer = lax.rem(my_y + dy, Y)
                recv = pltpu.make_async_remote_copy(
                    src_ref=mybest_ref,
                    dst_ref=gather_ref.at[peer],
                    send_sem=send_sems.at[dy - 1],
                    recv_sem=recv_sems.at[peer],
                    device_id=(my_x, peer, my_z),
                    device_id_type=pl.DeviceIdType.MESH,
                )
                recv.wait_recv()

            out_ref[...] = jnp.full((m, K), neg, jnp.float32)
            iota_k = lax.broadcasted_iota(jnp.int32, (m, K), 1)

            def extract2(k, carry):
                g = gather_ref[...]
                b = mybest_ref[...]
                mg = jnp.max(jnp.max(g, axis=2), axis=0)
                mb = jnp.max(b, axis=1)
                mx = jnp.maximum(mg, mb)
                out_ref[...] = jnp.where(iota_k == k, mx[:, None], out_ref[...])
                gather_ref[...] = jnp.where(g == mx[None, :, None], neg, g)
                mybest_ref[...] = jnp.where(b == mx[:, None], neg, b)
                return carry

            lax.fori_loop(0, K, extract2, 0)

    return pl.pallas_call(
        body,
        grid=(nsteps,),
        in_specs=[pl.BlockSpec((m, CHUNK), lambda i: (0, i))],
        out_specs=pl.BlockSpec((m, K), lambda i: (0, 0)),
        out_shape=jax.ShapeDtypeStruct((m, K), jnp.float32),
        scratch_shapes=[
            pltpu.VMEM((m, PAD), jnp.float32),
            pltpu.VMEM((m, PAD + CHUNK), jnp.float32),
            pltpu.VMEM((m, K), jnp.float32),
            pltpu.VMEM((Y, m, K), jnp.float32),
            pltpu.SemaphoreType.DMA((Y,)),
            pltpu.SemaphoreType.DMA((Y,)),
        ],
        compiler_params=pltpu.CompilerParams(collective_id=0),
    )(x)


# device time: 194871 ns/iter; 1.1454x vs baseline; 1.1454x over previous
import jax
import jax.numpy as jnp
from jax import lax
from jax.experimental import pallas as pl
from jax.experimental.pallas import tpu as pltpu

Y = 4
K = 32
CHUNK = 2048
PAD = 128


def kernel(x):
    m, n = x.shape
    nsteps = n // CHUNK
    neg = float("-inf")

    def body(x_ref, out_ref, best_ref, pool_ref, mybest_ref, gather_ref,
             send_sems, recv_sems):
        step = pl.program_id(0)
        my_x = lax.axis_index("x")
        my_y = lax.axis_index("y")
        my_z = lax.axis_index("z")

        @pl.when(step == 0)
        def _init():
            best_ref[...] = jnp.full((m, PAD), neg, jnp.float32)
            gather_ref[...] = jnp.full((Y, m, K), neg, jnp.float32)

        pool_ref[:, :PAD] = best_ref[...]
        pool_ref[:, PAD:] = x_ref[...]
        best_ref[...] = jnp.full((m, PAD), neg, jnp.float32)
        iota_pad = lax.broadcasted_iota(jnp.int32, (m, PAD), 1)

        def extract(k, prev):
            pv = jnp.where(pool_ref[...] == prev, neg, pool_ref[...])
            pool_ref[...] = pv
            mx = jnp.max(pv, axis=1, keepdims=True)
            best_ref[...] = jnp.where(iota_pad == k, mx, best_ref[...])
            return mx

        lax.fori_loop(0, K, extract, jnp.full((m, 1), jnp.inf, jnp.float32))

        @pl.when(step == nsteps - 1)
        def _comm():
            mybest_ref[...] = best_ref[:, :K]

            barrier = pltpu.get_barrier_semaphore()
            for dy in range(1, Y):
                peer = lax.rem(my_y + dy, Y)
                pl.semaphore_signal(
                    barrier, inc=1,
                    device_id=(my_x, peer, my_z),
                    device_id_type=pl.DeviceIdType.MESH,
                )
            pl.semaphore_wait(barrier, Y - 1)

            sends = []
            for dy in range(1, Y):
                peer = lax.rem(my_y + dy, Y)
                rdma = pltpu.make_async_remote_copy(
                    src_ref=mybest_ref,
                    dst_ref=gather_ref.at[my_y],
                    send_sem=send_sems.at[dy - 1],
                    recv_sem=recv_sems.at[my_y],
                    device_id=(my_x, peer, my_z),
                    device_id_type=pl.DeviceIdType.MESH,
                )
                rdma.start()
                sends.append(rdma)
            for rdma in sends:
                rdma.wait_send()
            for dy in range(1, Y):
                peer = lax.rem(my_y + dy, Y)
                recv = pltpu.make_async_remote_copy(
                    src_ref=mybest_ref,
                    dst_ref=gather_ref.at[peer],
                    send_sem=send_sems.at[dy - 1],
                    recv_sem=recv_sems.at[peer],
                    device_id=(my_x, peer, my_z),
                    device_id_type=pl.DeviceIdType.MESH,
                )
                recv.wait_recv()

            out_ref[...] = jnp.full((m, K), neg, jnp.float32)
            iota_k = lax.broadcasted_iota(jnp.int32, (m, K), 1)

            def extract2(k, carry):
                g = gather_ref[...]
                b = mybest_ref[...]
                mg = jnp.max(jnp.max(g, axis=2), axis=0)
                mb = jnp.max(b, axis=1)
                mx = jnp.maximum(mg, mb)
                out_ref[...] = jnp.where(iota_k == k, mx[:, None], out_ref[...])
                gather_ref[...] = jnp.where(g == mx[None, :, None], neg, g)
                mybest_ref[...] = jnp.where(b == mx[:, None], neg, b)
                return carry

            lax.fori_loop(0, K, extract2, 0)

    return pl.pallas_call(
        body,
        grid=(nsteps,),
        in_specs=[pl.BlockSpec((m, CHUNK), lambda i: (0, i))],
        out_specs=pl.BlockSpec((m, K), lambda i: (0, 0)),
        out_shape=jax.ShapeDtypeStruct((m, K), jnp.float32),
        scratch_shapes=[
            pltpu.VMEM((m, PAD), jnp.float32),
            pltpu.VMEM((m, PAD + CHUNK), jnp.float32),
            pltpu.VMEM((m, K), jnp.float32),
            pltpu.VMEM((Y, m, K), jnp.float32),
            pltpu.SemaphoreType.DMA((Y,)),
            pltpu.SemaphoreType.DMA((Y,)),
        ],
        compiler_params=pltpu.CompilerParams(collective_id=0),
    )(x)
